# baseline (speedup 1.0000x reference)
import jax
import jax.numpy as jnp
from jax.experimental import pallas as pl
from jax.experimental.pallas import tpu as pltpu

B, C, H, W = 32, 768, 32, 32
HW = H * W
CB = 256


def _select_body(m_ref, f_ref, t_ref, o_ref):
    o_ref[0] = jnp.where(m_ref[0] != 0, f_ref[0], t_ref[0])


def kernel(features, active_mask, mask_token):
    f3 = features.reshape(B, C, HW)
    m3 = active_mask.astype(jnp.int32).reshape(B, 1, HW)
    t3 = jnp.broadcast_to(mask_token.reshape(1, C, 1), (1, C, HW))
    out = pl.pallas_call(
        _select_body,
        grid=(C // CB, B),
        in_specs=[
            pl.BlockSpec((1, 1, HW), lambda c, b: (b, 0, 0)),
            pl.BlockSpec((1, CB, HW), lambda c, b: (b, c, 0)),
            pl.BlockSpec((1, CB, HW), lambda c, b: (0, c, 0)),
        ],
        out_specs=pl.BlockSpec((1, CB, HW), lambda c, b: (b, c, 0)),
        out_shape=jax.ShapeDtypeStruct((B, C, HW), jnp.float32),
        compiler_params=pltpu.CompilerParams(
            dimension_semantics=("parallel", "parallel"),
        ),
    )(m3, f3, t3)
    return out.reshape(B, C, H, W)
